# baseline (speedup 1.0000x reference)
import functools

import jax
import jax.numpy as jnp
from jax import lax
from jax.experimental import pallas as pl
from jax.experimental.pallas import tpu as pltpu
from jax.experimental.pallas import tpu_sc as plsc

NN = 10000
EE = 160000
NPAD = 10240
DIN = 256
HIDW = 128
NHEADS = 4
TWB = 160
TWI = TWB // 2
ADW = 16
BLK = 256
NBLK = NPAD // BLK

NTILE = 16
NCORE = 2
ROWS_PER_TILE = NPAD // NTILE
EB1 = 80
NB1 = (EE // NTILE) // EB1
EB2 = 40
NB2 = (EE // (NTILE * NCORE)) // EB2
EIDX_PAD = EE + 4 * EB1
MHI = -65536


def _l1_tables_body(x_ref, w1_ref, a1s_ref, a1d_ref, t_ref, ad_ref):
    h = jnp.dot(x_ref[...], w1_ref[...], preferred_element_type=jnp.float32)
    ad_ref[:, NHEADS:ADW] = jnp.zeros((BLK, ADW - NHEADS), jnp.float32)
    for hd in range(NHEADS):
        hh = h[:, hd * HIDW:(hd + 1) * HIDW]
        asv = jnp.sum(hh * a1s_ref[hd, :][None, :], axis=1, keepdims=True)
        adv = jnp.sum(hh * a1d_ref[hd, :][None, :], axis=1, keepdims=True)
        t_ref[hd, :, 0:HIDW] = hh.astype(jnp.bfloat16)
        t_ref[hd, :, HIDW:HIDW + 1] = jnp.ones((BLK, 1), jnp.bfloat16)
        t_ref[hd, :, HIDW + 1:HIDW + 2] = asv.astype(jnp.bfloat16)
        t_ref[hd, :, HIDW + 2:TWB] = jnp.zeros((BLK, TWB - HIDW - 2),
                                               jnp.bfloat16)
        ad_ref[:, hd:hd + 1] = adv


def _l1_tables(x_pad, W1, a1_src, a1_dst):
    return pl.pallas_call(
        _l1_tables_body,
        grid=(NBLK,),
        in_specs=[
            pl.BlockSpec((BLK, DIN), lambda i: (i, 0)),
            pl.BlockSpec((DIN, NHEADS * HIDW), lambda i: (0, 0)),
            pl.BlockSpec((NHEADS, HIDW), lambda i: (0, 0)),
            pl.BlockSpec((NHEADS, HIDW), lambda i: (0, 0)),
        ],
        out_specs=[
            pl.BlockSpec((NHEADS, BLK, TWB), lambda i: (0, i, 0)),
            pl.BlockSpec((BLK, ADW), lambda i: (i, 0)),
        ],
        out_shape=[
            jax.ShapeDtypeStruct((NHEADS, NPAD, TWB), jnp.bfloat16),
            jax.ShapeDtypeStruct((NPAD, ADW), jnp.float32),
        ],
    )(x_pad, W1, a1_src, a1_dst)


def _l2_tables_body(acc_ref, den_ref, b1_ref, w2_ref, a2s_ref, a2d_ref,
                    t_ref, ad_ref):
    hs = []
    for hd in range(NHEADS):
        num = acc_ref[hd]
        den = den_ref[hd]
        v = num / (den + 1e-16) + b1_ref[hd, :][None, :]
        hs.append(jnp.where(v > 0, v, jnp.exp(v) - 1.0))
    h1n = jnp.concatenate(hs, axis=1)
    h2 = jnp.dot(h1n, w2_ref[...], preferred_element_type=jnp.float32)
    asv = jnp.sum(h2 * a2s_ref[0, :][None, :], axis=1, keepdims=True)
    adv = jnp.sum(h2 * a2d_ref[0, :][None, :], axis=1, keepdims=True)
    t_ref[:, 0:HIDW] = h2.astype(jnp.bfloat16)
    t_ref[:, HIDW:HIDW + 1] = jnp.ones((BLK, 1), jnp.bfloat16)
    t_ref[:, HIDW + 1:HIDW + 2] = asv.astype(jnp.bfloat16)
    t_ref[:, HIDW + 2:TWB] = jnp.zeros((BLK, TWB - HIDW - 2), jnp.bfloat16)
    ad_ref[:, 0:1] = adv
    ad_ref[:, 1:ADW] = jnp.zeros((BLK, ADW - 1), jnp.float32)


def _l2_tables(acc1, den1, b1r, W2, a2_src, a2_dst):
    return pl.pallas_call(
        _l2_tables_body,
        grid=(NBLK,),
        in_specs=[
            pl.BlockSpec((NHEADS, BLK, HIDW), lambda i: (0, i, 0)),
            pl.BlockSpec((NHEADS, BLK, 1), lambda i: (0, i, 0)),
            pl.BlockSpec((NHEADS, HIDW), lambda i: (0, 0)),
            pl.BlockSpec((NHEADS * HIDW, HIDW), lambda i: (0, 0)),
            pl.BlockSpec((1, HIDW), lambda i: (0, 0)),
            pl.BlockSpec((1, HIDW), lambda i: (0, 0)),
        ],
        out_specs=[
            pl.BlockSpec((BLK, TWB), lambda i: (i, 0)),
            pl.BlockSpec((BLK, ADW), lambda i: (i, 0)),
        ],
        out_shape=[
            jax.ShapeDtypeStruct((NPAD, TWB), jnp.bfloat16),
            jax.ShapeDtypeStruct((NPAD, ADW), jnp.float32),
        ],
    )(acc1, den1, b1r, W2, a2_src, a2_dst)


def _final_body(acc_ref, den_ref, b2_ref, wout_ref, bout_ref, o_ref):
    a = acc_ref[0] + acc_ref[1]
    d = den_ref[0] + den_ref[1]
    v = a / (d + 1e-16) + b2_ref[0, :][None, :]
    h = jnp.where(v > 0, v, jnp.exp(v) - 1.0)
    o_ref[...] = jnp.dot(h, wout_ref[...],
                         preferred_element_type=jnp.float32) + bout_ref[0, :][None, :]


def _final(acc2, den2, b2r, Wout, boutr):
    return pl.pallas_call(
        _final_body,
        grid=(NBLK,),
        in_specs=[
            pl.BlockSpec((NCORE, BLK, HIDW), lambda i: (0, i, 0)),
            pl.BlockSpec((NCORE, BLK, 1), lambda i: (0, i, 0)),
            pl.BlockSpec((1, HIDW), lambda i: (0, 0)),
            pl.BlockSpec((HIDW, 1), lambda i: (0, 0)),
            pl.BlockSpec((1, 1), lambda i: (0, 0)),
        ],
        out_specs=pl.BlockSpec((BLK, 1), lambda i: (i, 0)),
        out_shape=jax.ShapeDtypeStruct((NPAD, 1), jnp.float32),
    )(acc2, den2, b2r, Wout, boutr)


ZROWS = 32


def _w_offsets(nb):
    offs = list(range(0, nb - 15, 16))
    if offs[-1] != nb - 16:
        offs.append(nb - 16)
    return offs


def _zero_fill(zero_v, nrows):
    zv = jnp.zeros((16,), jnp.float32)

    def zrow(j, _):
        for k in range(HIDW // 16):
            zero_v[j, pl.ds(k * 16, 16)] = zv
        return 0

    lax.fori_loop(0, nrows, zrow, 0)


def _unpack_scale(rbuf, obuf, w_v, eb):
    i16 = lax.iota(jnp.int32, 16)
    ev = i16 * 2

    def group(g, w16, r0):
        for j in range(16):
            wr = w16[j]
            rfull = i16 * 0 + (r0 + j)
            for k in range(HIDW // 32):
                wd = rbuf[r0 + j, pl.ds(k * 16, 16)]
                lo = plsc.bitcast(wd << 16, jnp.float32) * wr
                hi = plsc.bitcast(wd & MHI, jnp.float32) * wr
                plsc.store_scatter(obuf, [rfull, ev + k * 32], lo)
                plsc.store_scatter(obuf, [rfull, ev + (k * 32 + 1)], hi)

    def body(g, _):
        group(g, w_v[pl.ds(g * 16, 16)], g * 16)
        return 0

    lax.fori_loop(0, eb // 16, body, 0)
    rem = eb - (eb // 16) * 16
    if rem:
        r0 = (eb // 16) * 16
        lane0 = r0 - (eb - 16)
        w16 = w_v[pl.ds(eb - 16, 16)]
        i16_ = lax.iota(jnp.int32, 16)
        for j in range(rem):
            wr = w16[lane0 + j]
            rfull = i16_ * 0 + (r0 + j)
            for k in range(HIDW // 32):
                wd = rbuf[r0 + j, pl.ds(k * 16, 16)]
                lo = plsc.bitcast(wd << 16, jnp.float32) * wr
                hi = plsc.bitcast(wd & MHI, jnp.float32) * wr
                plsc.store_scatter(obuf, [rfull, i16_ * 2 + k * 32], lo)
                plsc.store_scatter(obuf, [rfull, i16_ * 2 + (k * 32 + 1)], hi)


def _pipe_step(ctx, i, b, eb, row_off, adcol, first):
    (t_hbm, ad_hbm, src_hbm, dst_hbm, acc_sp, den_sp, sbuf, dbuf, scat, rbuf,
     obuf, abuf, wbuf, semi, semg, sems, tile_base) = ctx
    b2 = 1 - b
    i16 = lax.iota(jnp.int32, 16)
    cw = i16 * 0 + (HIDW // 2)
    cad = i16 * 0 + adcol
    pltpu.make_async_copy(t_hbm.at[sbuf[b]], rbuf[b], semg[b]).wait()
    pltpu.make_async_copy(ad_hbm.at[dbuf[b]], abuf[b], semg[b]).wait()
    for off in _w_offsets(eb):
        scat[b][pl.ds(off, 16)] = dbuf[b][pl.ds(off, 16)]
    base2 = tile_base + (i + 2) * eb
    pltpu.async_copy(src_hbm.at[pl.ds(base2, eb)], sbuf[b], semi[b])
    pltpu.async_copy(dst_hbm.at[pl.ds(base2, eb)], dbuf[b], semi[b])
    base1 = tile_base + (i + 1) * eb
    pltpu.make_async_copy(src_hbm.at[pl.ds(base1, eb)], sbuf[b2], semi[b2]).wait()
    pltpu.make_async_copy(dst_hbm.at[pl.ds(base1, eb)], dbuf[b2], semi[b2]).wait()
    if row_off is not None:
        for off in range(0, eb, 16):
            sbuf[b2][pl.ds(off, 16)] = sbuf[b2][pl.ds(off, 16)] + row_off
    if not first:
        pltpu.make_async_copy(obuf[b2], acc_sp.at[scat[b2]], sems[b2]).wait()
        pltpu.make_async_copy(wbuf[b2], den_sp.at[scat[b2]], sems[b2]).wait()
    pltpu.async_copy(t_hbm.at[sbuf[b2]], rbuf[b2], semg[b2])
    pltpu.async_copy(ad_hbm.at[dbuf[b2]], abuf[b2], semg[b2])
    for off in _w_offsets(eb):
        rr = i16 + off
        wd = plsc.load_gather(rbuf[b], [rr, cw])
        asg = plsc.bitcast(wd & MHI, jnp.float32)
        adg = plsc.load_gather(abuf[b], [rr, cad])
        e = asg + adg
        e = jnp.maximum(e, e * 0.2)
        wbuf[b][pl.ds(off, 16)] = jnp.exp(e)
    _unpack_scale(rbuf[b], obuf[b], wbuf[b], eb)
    pltpu.async_copy(obuf[b], acc_sp.at[scat[b]], sems[b], add=True)
    pltpu.async_copy(wbuf[b], den_sp.at[scat[b]], sems[b], add=True)


def _edge_pass(ctx, nb, eb, row_off, adcol):
    (t_hbm, ad_hbm, src_hbm, dst_hbm, acc_sp, den_sp, sbuf, dbuf, scat, rbuf,
     obuf, abuf, wbuf, semi, semg, sems, tile_base) = ctx
    assert (nb - 1) % 2 == 0
    for j in range(2):
        base = tile_base + j * eb
        pltpu.async_copy(src_hbm.at[pl.ds(base, eb)], sbuf[j], semi[j])
        pltpu.async_copy(dst_hbm.at[pl.ds(base, eb)], dbuf[j], semi[j])
    pltpu.make_async_copy(src_hbm.at[pl.ds(tile_base, eb)], sbuf[0], semi[0]).wait()
    pltpu.make_async_copy(dst_hbm.at[pl.ds(tile_base, eb)], dbuf[0], semi[0]).wait()
    if row_off is not None:
        for off in range(0, eb, 16):
            sbuf[0][pl.ds(off, 16)] = sbuf[0][pl.ds(off, 16)] + row_off
    pltpu.async_copy(t_hbm.at[sbuf[0]], rbuf[0], semg[0])
    pltpu.async_copy(ad_hbm.at[dbuf[0]], abuf[0], semg[0])
    _pipe_step(ctx, 0, 0, eb, row_off, adcol, True)

    def gbody(g, _):
        _pipe_step(ctx, 2 * g + 1, 1, eb, row_off, adcol, False)
        _pipe_step(ctx, 2 * g + 2, 0, eb, row_off, adcol, False)
        return 0

    lax.fori_loop(0, (nb - 1) // 2, gbody, 0)
    bg = nb % 2
    bs = (nb - 1) % 2
    pltpu.make_async_copy(t_hbm.at[sbuf[bg]], rbuf[bg], semg[bg]).wait()
    pltpu.make_async_copy(ad_hbm.at[dbuf[bg]], abuf[bg], semg[bg]).wait()
    pltpu.make_async_copy(obuf[bs], acc_sp.at[scat[bs]], sems[bs]).wait()
    pltpu.make_async_copy(wbuf[bs], den_sp.at[scat[bs]], sems[bs]).wait()
    base = tile_base + (nb + 1) * eb
    pltpu.make_async_copy(src_hbm.at[pl.ds(base, eb)], sbuf[bs], semi[bs]).wait()
    pltpu.make_async_copy(dst_hbm.at[pl.ds(base, eb)], dbuf[bs], semi[bs]).wait()


def _acc_zero(acc_sp, den_sp, zero_v, s):
    def zacc(j, _):
        pltpu.sync_copy(zero_v,
                        acc_sp.at[pl.ds(s * ROWS_PER_TILE + j * ZROWS, ZROWS)])
        return 0
    lax.fori_loop(0, ROWS_PER_TILE // ZROWS, zacc, 0)

    def zden(j, _):
        pltpu.sync_copy(zero_v.at[0],
                        den_sp.at[pl.ds(s * ROWS_PER_TILE + j * HIDW, HIDW)])
        return 0
    lax.fori_loop(0, ROWS_PER_TILE // HIDW, zden, 0)


def _acc_readout(acc_sp, den_sp, stage_v, dstage_v, acc_hbm, den_hbm, s,
                 out_row_base):
    def rdout(j, _):
        rs = s * ROWS_PER_TILE + j * ZROWS
        pltpu.sync_copy(acc_sp.at[pl.ds(rs, ZROWS)], stage_v)
        pltpu.sync_copy(stage_v, acc_hbm.at[pl.ds(out_row_base + rs, ZROWS)])
        return 0
    lax.fori_loop(0, ROWS_PER_TILE // ZROWS, rdout, 0)

    def rden(j, _):
        rs = s * ROWS_PER_TILE + j * HIDW
        pltpu.sync_copy(den_sp.at[pl.ds(rs, HIDW)], dstage_v)
        pltpu.sync_copy(dstage_v, den_hbm.at[pl.ds(out_row_base + rs, HIDW)])
        return 0
    lax.fori_loop(0, ROWS_PER_TILE // HIDW, rden, 0)


def _sc_scratch(eb):
    return [
        pltpu.VMEM_SHARED((NPAD, HIDW), jnp.float32),
        pltpu.VMEM_SHARED((NPAD,), jnp.float32),
        pltpu.VMEM((eb,), jnp.int32),
        pltpu.VMEM((eb,), jnp.int32),
        pltpu.VMEM((eb,), jnp.int32),
        pltpu.VMEM((eb,), jnp.int32),
        pltpu.VMEM((eb,), jnp.int32),
        pltpu.VMEM((eb,), jnp.int32),
        pltpu.VMEM((eb, TWI), jnp.int32),
        pltpu.VMEM((eb, TWI), jnp.int32),
        pltpu.VMEM((eb, HIDW), jnp.float32),
        pltpu.VMEM((eb, HIDW), jnp.float32),
        pltpu.VMEM((eb, ADW), jnp.float32),
        pltpu.VMEM((eb, ADW), jnp.float32),
        pltpu.VMEM((eb,), jnp.float32),
        pltpu.VMEM((eb,), jnp.float32),
        pltpu.VMEM((ZROWS, HIDW), jnp.float32),
        pltpu.VMEM((HIDW,), jnp.float32),
        pltpu.SemaphoreType.DMA,
        pltpu.SemaphoreType.DMA,
        pltpu.SemaphoreType.DMA,
        pltpu.SemaphoreType.DMA,
        pltpu.SemaphoreType.DMA,
        pltpu.SemaphoreType.DMA,
    ]


def _sc_layer1(tflat, adtab, src, dst):
    mesh = plsc.VectorSubcoreMesh(core_axis_name="c", subcore_axis_name="s")

    @functools.partial(
        pl.kernel,
        out_type=[
            jax.ShapeDtypeStruct((NHEADS * NPAD, HIDW), jnp.float32),
            jax.ShapeDtypeStruct((NHEADS * NPAD,), jnp.float32),
        ],
        mesh=mesh,
        compiler_params=pltpu.CompilerParams(use_tc_tiling_on_sc=False,
                                             needs_layout_passes=False),
        scratch_types=_sc_scratch(EB1),
    )
    def body(t_hbm, ad_hbm, src_hbm, dst_hbm, acc_hbm, den_hbm, acc_sp,
             den_sp, s0, s1, d0, d1, x0, x1, r0, r1, o0, o1, a0, a1, w0, w1,
             stage_v, dstage_v, si0, si1, sg0, sg1, ss0, ss1):
        c = lax.axis_index("c")
        s = lax.axis_index("s")
        _zero_fill(stage_v, ZROWS)
        ctx = (t_hbm, ad_hbm, src_hbm, dst_hbm, acc_sp, den_sp, (s0, s1),
               (d0, d1), (x0, x1), (r0, r1), (o0, o1), (a0, a1), (w0, w1),
               (si0, si1), (sg0, sg1), (ss0, ss1), s * (EE // NTILE))
        for p in range(2):
            hd = 2 * c + p
            _acc_zero(acc_sp, den_sp, stage_v, s)
            plsc.subcore_barrier()
            _edge_pass(ctx, NB1, EB1, hd * NPAD, hd)
            plsc.subcore_barrier()
            _acc_readout(acc_sp, den_sp, stage_v, dstage_v, acc_hbm, den_hbm,
                         s, hd * NPAD)
            plsc.subcore_barrier()
            if p == 0:
                _zero_fill(stage_v, ZROWS)

    return body(tflat, adtab, src, dst)


def _sc_layer2(tbl, adtab, src, dst):
    mesh = plsc.VectorSubcoreMesh(core_axis_name="c", subcore_axis_name="s")

    @functools.partial(
        pl.kernel,
        out_type=[
            jax.ShapeDtypeStruct((NCORE * NPAD, HIDW), jnp.float32),
            jax.ShapeDtypeStruct((NCORE * NPAD,), jnp.float32),
        ],
        mesh=mesh,
        compiler_params=pltpu.CompilerParams(use_tc_tiling_on_sc=False,
                                             needs_layout_passes=False),
        scratch_types=_sc_scratch(EB2),
    )
    def body(t_hbm, ad_hbm, src_hbm, dst_hbm, acc_hbm, den_hbm, acc_sp,
             den_sp, s0, s1, d0, d1, x0, x1, r0, r1, o0, o1, a0, a1, w0, w1,
             stage_v, dstage_v, si0, si1, sg0, sg1, ss0, ss1):
        c = lax.axis_index("c")
        s = lax.axis_index("s")
        wid = c * NTILE + s
        _zero_fill(stage_v, ZROWS)
        _acc_zero(acc_sp, den_sp, stage_v, s)
        plsc.subcore_barrier()
        ctx = (t_hbm, ad_hbm, src_hbm, dst_hbm, acc_sp, den_sp, (s0, s1),
               (d0, d1), (x0, x1), (r0, r1), (o0, o1), (a0, a1), (w0, w1),
               (si0, si1), (sg0, sg1), (ss0, ss1),
               wid * (EE // (NTILE * NCORE)))
        _edge_pass(ctx, NB2, EB2, None, 0)
        plsc.subcore_barrier()
        _acc_readout(acc_sp, den_sp, stage_v, dstage_v, acc_hbm, den_hbm, s,
                     c * NPAD)

    return body(tbl, adtab, src, dst)


def kernel(x, edge_index, W1, a1_src, a1_dst, b1, W2, a2_src, a2_dst, b2,
           Wout, bout):
    x_pad = jnp.pad(x, ((0, NPAD - NN), (0, 0)))
    src = jnp.pad(edge_index[0], (0, EIDX_PAD - EE))
    dst = jnp.pad(edge_index[1], (0, EIDX_PAD - EE))

    table1, adtab1 = _l1_tables(x_pad, W1, a1_src, a1_dst)
    t1w = lax.bitcast_convert_type(
        table1.reshape(NHEADS * NPAD, TWI, 2), jnp.int32)
    acc1, den1 = _sc_layer1(t1w, adtab1, src, dst)
    acc1 = acc1.reshape(NHEADS, NPAD, HIDW)
    den1 = den1.reshape(NHEADS, NPAD, 1)

    table2, adtab2 = _l2_tables(acc1, den1, b1.reshape(NHEADS, HIDW), W2,
                                a2_src, a2_dst)
    t2w = lax.bitcast_convert_type(table2.reshape(NPAD, TWI, 2), jnp.int32)
    acc2, den2 = _sc_layer2(t2w, adtab2, src, dst)
    acc2 = acc2.reshape(NCORE, NPAD, HIDW)
    den2 = den2.reshape(NCORE, NPAD, 1)

    y = _final(acc2, den2, b2.reshape(1, HIDW), Wout, bout.reshape(1, 1))
    return y[:NN]


# speedup vs baseline: 2.1332x; 2.1332x over previous
import functools

import jax
import jax.numpy as jnp
from jax import lax
from jax.experimental import pallas as pl
from jax.experimental.pallas import tpu as pltpu
from jax.experimental.pallas import tpu_sc as plsc

NN = 10000
EE = 160000
NPAD = 10240
DIN = 256
HIDW = 128
NHEADS = 4
ADW = 16
BLK = 256
NBLK = NPAD // BLK

NTILE = 16
NCORE = 2
ROWS_PER_TILE = NPAD // NTILE
EB1 = 80
NB1 = (EE // NTILE) // EB1
EB2 = 40
NB2 = (EE // (NTILE * NCORE)) // EB2
EIDX_PAD = EE + 4 * EB1
ZROWS = 20


def _l1_tables_body(x_ref, w1_ref, a1s_ref, a1d_ref, t_ref, as_ref, ad_ref):
    h = jnp.dot(x_ref[...], w1_ref[...], preferred_element_type=jnp.float32)
    as_ref[:, NHEADS:ADW] = jnp.zeros((BLK, ADW - NHEADS), jnp.float32)
    ad_ref[:, NHEADS:ADW] = jnp.zeros((BLK, ADW - NHEADS), jnp.float32)
    for hd in range(NHEADS):
        hh = h[:, hd * HIDW:(hd + 1) * HIDW]
        asv = jnp.sum(hh * a1s_ref[hd, :][None, :], axis=1, keepdims=True)
        adv = jnp.sum(hh * a1d_ref[hd, :][None, :], axis=1, keepdims=True)
        t_ref[hd] = hh
        as_ref[:, hd:hd + 1] = asv
        ad_ref[:, hd:hd + 1] = adv


def _l1_tables(x_pad, W1, a1_src, a1_dst):
    return pl.pallas_call(
        _l1_tables_body,
        grid=(NBLK,),
        in_specs=[
            pl.BlockSpec((BLK, DIN), lambda i: (i, 0)),
            pl.BlockSpec((DIN, NHEADS * HIDW), lambda i: (0, 0)),
            pl.BlockSpec((NHEADS, HIDW), lambda i: (0, 0)),
            pl.BlockSpec((NHEADS, HIDW), lambda i: (0, 0)),
        ],
        out_specs=[
            pl.BlockSpec((NHEADS, BLK, HIDW), lambda i: (0, i, 0)),
            pl.BlockSpec((BLK, ADW), lambda i: (i, 0)),
            pl.BlockSpec((BLK, ADW), lambda i: (i, 0)),
        ],
        out_shape=[
            jax.ShapeDtypeStruct((NHEADS, NPAD, HIDW), jnp.float32),
            jax.ShapeDtypeStruct((NPAD, ADW), jnp.float32),
            jax.ShapeDtypeStruct((NPAD, ADW), jnp.float32),
        ],
    )(x_pad, W1, a1_src, a1_dst)


def _l2_tables_body(acc_ref, den_ref, b1_ref, w2_ref, a2s_ref, a2d_ref,
                    t_ref, as_ref, ad_ref):
    hs = []
    for hd in range(NHEADS):
        v = acc_ref[hd] / (den_ref[hd] + 1e-16) + b1_ref[hd, :][None, :]
        hs.append(jnp.where(v > 0, v, jnp.exp(v) - 1.0))
    h1n = jnp.concatenate(hs, axis=1)
    h2 = jnp.dot(h1n, w2_ref[...], preferred_element_type=jnp.float32)
    asv = jnp.sum(h2 * a2s_ref[0, :][None, :], axis=1, keepdims=True)
    adv = jnp.sum(h2 * a2d_ref[0, :][None, :], axis=1, keepdims=True)
    t_ref[...] = h2
    as_ref[:, 0:1] = asv
    as_ref[:, 1:ADW] = jnp.zeros((BLK, ADW - 1), jnp.float32)
    ad_ref[:, 0:1] = adv
    ad_ref[:, 1:ADW] = jnp.zeros((BLK, ADW - 1), jnp.float32)


def _l2_tables(acc1, den1, b1r, W2, a2_src, a2_dst):
    return pl.pallas_call(
        _l2_tables_body,
        grid=(NBLK,),
        in_specs=[
            pl.BlockSpec((NHEADS, BLK, HIDW), lambda i: (0, i, 0)),
            pl.BlockSpec((NHEADS, BLK, 1), lambda i: (0, i, 0)),
            pl.BlockSpec((NHEADS, HIDW), lambda i: (0, 0)),
            pl.BlockSpec((NHEADS * HIDW, HIDW), lambda i: (0, 0)),
            pl.BlockSpec((1, HIDW), lambda i: (0, 0)),
            pl.BlockSpec((1, HIDW), lambda i: (0, 0)),
        ],
        out_specs=[
            pl.BlockSpec((BLK, HIDW), lambda i: (i, 0)),
            pl.BlockSpec((BLK, ADW), lambda i: (i, 0)),
            pl.BlockSpec((BLK, ADW), lambda i: (i, 0)),
        ],
        out_shape=[
            jax.ShapeDtypeStruct((NPAD, HIDW), jnp.float32),
            jax.ShapeDtypeStruct((NPAD, ADW), jnp.float32),
            jax.ShapeDtypeStruct((NPAD, ADW), jnp.float32),
        ],
    )(acc1, den1, b1r, W2, a2_src, a2_dst)


def _final_body(acc_ref, den_ref, b2_ref, wout_ref, bout_ref, o_ref):
    a = acc_ref[0] + acc_ref[1]
    d = den_ref[0] + den_ref[1]
    v = a / (d + 1e-16) + b2_ref[0, :][None, :]
    h = jnp.where(v > 0, v, jnp.exp(v) - 1.0)
    o_ref[...] = jnp.dot(h, wout_ref[...],
                         preferred_element_type=jnp.float32) + bout_ref[0, :][None, :]


def _final(acc2, den2, b2r, Wout, boutr):
    return pl.pallas_call(
        _final_body,
        grid=(NBLK,),
        in_specs=[
            pl.BlockSpec((NCORE, BLK, HIDW), lambda i: (0, i, 0)),
            pl.BlockSpec((NCORE, BLK, 1), lambda i: (0, i, 0)),
            pl.BlockSpec((1, HIDW), lambda i: (0, 0)),
            pl.BlockSpec((HIDW, 1), lambda i: (0, 0)),
            pl.BlockSpec((1, 1), lambda i: (0, 0)),
        ],
        out_specs=pl.BlockSpec((BLK, 1), lambda i: (i, 0)),
        out_shape=jax.ShapeDtypeStruct((NPAD, 1), jnp.float32),
    )(acc2, den2, b2r, Wout, boutr)


def _w_offsets(nb):
    offs = list(range(0, nb - 15, 16))
    if offs[-1] != nb - 16:
        offs.append(nb - 16)
    return offs


def _zero_fill(zero_v, nrows):
    zv = jnp.zeros((16,), jnp.float32)

    def zrow(j, _):
        for k in range(HIDW // 16):
            zero_v[j, pl.ds(k * 16, 16)] = zv
        return 0

    lax.fori_loop(0, nrows, zrow, 0)


def _scale_rows(rows, w_v, eb):
    def group(w16, r0, j0, nj):
        for j in range(j0, nj):
            wr = w16[j]
            for k in range(HIDW // 16):
                rows[r0 + j, pl.ds(k * 16, 16)] = (
                    rows[r0 + j, pl.ds(k * 16, 16)] * wr)

    def body(g, _):
        group(w_v[pl.ds(g * 16, 16)], g * 16, 0, 16)
        return 0

    lax.fori_loop(0, eb // 16, body, 0)
    rem = eb - (eb // 16) * 16
    if rem:
        lane0 = (eb // 16) * 16 - (eb - 16)
        w16 = w_v[pl.ds(eb - 16, 16)]
        r0 = (eb // 16) * 16
        for j in range(rem):
            wr = w16[lane0 + j]
            for k in range(HIDW // 16):
                rows[r0 + j, pl.ds(k * 16, 16)] = (
                    rows[r0 + j, pl.ds(k * 16, 16)] * wr)


def _issue_gathers(ctx, i, b, eb, row_off, adcol, wait_scatter):
    (t_hbm, as_hbm, ad_hbm, src_hbm, dst_hbm, acc_sp, den_sp, sbuf, dbuf,
     scat, rbuf, abuf, ebuf, wbuf, semi, semg, sems, tile_base) = ctx
    base = tile_base + i * eb
    pltpu.make_async_copy(src_hbm.at[pl.ds(base, eb)], sbuf[b], semi[b]).wait()
    pltpu.make_async_copy(dst_hbm.at[pl.ds(base, eb)], dbuf[b], semi[b]).wait()
    if wait_scatter:
        pltpu.make_async_copy(rbuf[b], acc_sp.at[scat[b]], sems[b]).wait()
        pltpu.make_async_copy(wbuf[b], den_sp.at[scat[b]], sems[b]).wait()
    pltpu.async_copy(as_hbm.at[sbuf[b]], abuf[b], semg[b])
    if row_off is not None:
        for off in range(0, eb, 16):
            sbuf[b][pl.ds(off, 16)] = sbuf[b][pl.ds(off, 16)] + row_off
    pltpu.async_copy(t_hbm.at[sbuf[b]], rbuf[b], semg[b])
    pltpu.async_copy(ad_hbm.at[dbuf[b]], ebuf[b], semg[b])


def _pipe_step(ctx, i, b, eb, row_off, adcol, first, second):
    (t_hbm, as_hbm, ad_hbm, src_hbm, dst_hbm, acc_sp, den_sp, sbuf, dbuf,
     scat, rbuf, abuf, ebuf, wbuf, semi, semg, sems, tile_base) = ctx
    b2 = (b + 2) % 3
    i16 = lax.iota(jnp.int32, 16)
    cas = i16 * 0 + adcol
    pltpu.make_async_copy(as_hbm.at[sbuf[b]], abuf[b], semg[b]).wait()
    pltpu.make_async_copy(t_hbm.at[sbuf[b]], rbuf[b], semg[b]).wait()
    pltpu.make_async_copy(ad_hbm.at[dbuf[b]], ebuf[b], semg[b]).wait()
    for off in _w_offsets(eb):
        scat[b][pl.ds(off, 16)] = dbuf[b][pl.ds(off, 16)]
    base3 = tile_base + (i + 3) * eb
    pltpu.async_copy(src_hbm.at[pl.ds(base3, eb)], sbuf[b], semi[b])
    pltpu.async_copy(dst_hbm.at[pl.ds(base3, eb)], dbuf[b], semi[b])
    for off in _w_offsets(eb):
        rr = i16 + off
        asg = plsc.load_gather(abuf[b], [rr, cas])
        adg = plsc.load_gather(ebuf[b], [rr, cas])
        e = asg + adg
        e = jnp.maximum(e, e * 0.2)
        wbuf[b][pl.ds(off, 16)] = jnp.exp(e)
    _scale_rows(rbuf[b], wbuf[b], eb)
    pltpu.async_copy(rbuf[b], acc_sp.at[scat[b]], sems[b], add=True)
    pltpu.async_copy(wbuf[b], den_sp.at[scat[b]], sems[b], add=True)
    _issue_gathers(ctx, i + 2, b2, eb, row_off, adcol, not first)


def _edge_pass(ctx, nb, eb, row_off, adcol):
    (t_hbm, as_hbm, ad_hbm, src_hbm, dst_hbm, acc_sp, den_sp, sbuf, dbuf,
     scat, rbuf, abuf, ebuf, wbuf, semi, semg, sems, tile_base) = ctx
    assert (nb - 2) % 3 == 0
    for j in range(3):
        base = tile_base + j * eb
        pltpu.async_copy(src_hbm.at[pl.ds(base, eb)], sbuf[j], semi[j])
        pltpu.async_copy(dst_hbm.at[pl.ds(base, eb)], dbuf[j], semi[j])
    _issue_gathers(ctx, 0, 0, eb, row_off, adcol, False)
    _issue_gathers(ctx, 1, 1, eb, row_off, adcol, False)
    _pipe_step(ctx, 0, 0, eb, row_off, adcol, True, False)
    _pipe_step(ctx, 1, 1, eb, row_off, adcol, False, True)

    def gbody(g, _):
        i = 3 * g + 2
        _pipe_step(ctx, i, 2, eb, row_off, adcol, False, False)
        _pipe_step(ctx, i + 1, 0, eb, row_off, adcol, False, False)
        _pipe_step(ctx, i + 2, 1, eb, row_off, adcol, False, False)
        return 0

    lax.fori_loop(0, (nb - 2) // 3, gbody, 0)
    for i in (nb, nb + 1):
        b = i % 3
        pltpu.make_async_copy(as_hbm.at[sbuf[b]], abuf[b], semg[b]).wait()
        pltpu.make_async_copy(t_hbm.at[sbuf[b]], rbuf[b], semg[b]).wait()
        pltpu.make_async_copy(ad_hbm.at[dbuf[b]], ebuf[b], semg[b]).wait()
    b = (nb - 1) % 3
    pltpu.make_async_copy(rbuf[b], acc_sp.at[scat[b]], sems[b]).wait()
    pltpu.make_async_copy(wbuf[b], den_sp.at[scat[b]], sems[b]).wait()
    i = nb + 2
    b = i % 3
    base = tile_base + i * eb
    pltpu.make_async_copy(src_hbm.at[pl.ds(base, eb)], sbuf[b], semi[b]).wait()
    pltpu.make_async_copy(dst_hbm.at[pl.ds(base, eb)], dbuf[b], semi[b]).wait()


def _acc_zero(acc_sp, den_sp, zero_v, s):
    def zacc(j, _):
        pltpu.sync_copy(zero_v,
                        acc_sp.at[pl.ds(s * ROWS_PER_TILE + j * ZROWS, ZROWS)])
        return 0
    lax.fori_loop(0, ROWS_PER_TILE // ZROWS, zacc, 0)

    def zden(j, _):
        pltpu.sync_copy(zero_v.at[0],
                        den_sp.at[pl.ds(s * ROWS_PER_TILE + j * HIDW, HIDW)])
        return 0
    lax.fori_loop(0, ROWS_PER_TILE // HIDW, zden, 0)


def _acc_readout(acc_sp, den_sp, stage_v, dstage_v, acc_hbm, den_hbm, s,
                 out_row_base):
    def rdout(j, _):
        rs = s * ROWS_PER_TILE + j * ZROWS
        pltpu.sync_copy(acc_sp.at[pl.ds(rs, ZROWS)], stage_v)
        pltpu.sync_copy(stage_v, acc_hbm.at[pl.ds(out_row_base + rs, ZROWS)])
        return 0
    lax.fori_loop(0, ROWS_PER_TILE // ZROWS, rdout, 0)

    def rden(j, _):
        rs = s * ROWS_PER_TILE + j * HIDW
        pltpu.sync_copy(den_sp.at[pl.ds(rs, HIDW)], dstage_v)
        pltpu.sync_copy(dstage_v, den_hbm.at[pl.ds(out_row_base + rs, HIDW)])
        return 0
    lax.fori_loop(0, ROWS_PER_TILE // HIDW, rden, 0)


def _sc_scratch(eb):
    return [
        pltpu.VMEM_SHARED((NPAD, HIDW), jnp.float32),
        pltpu.VMEM_SHARED((NPAD,), jnp.float32),
        pltpu.VMEM((eb,), jnp.int32),
        pltpu.VMEM((eb,), jnp.int32),
        pltpu.VMEM((eb,), jnp.int32),
        pltpu.VMEM((eb,), jnp.int32),
        pltpu.VMEM((eb,), jnp.int32),
        pltpu.VMEM((eb,), jnp.int32),
        pltpu.VMEM((eb,), jnp.int32),
        pltpu.VMEM((eb,), jnp.int32),
        pltpu.VMEM((eb,), jnp.int32),
        pltpu.VMEM((eb, HIDW), jnp.float32),
        pltpu.VMEM((eb, HIDW), jnp.float32),
        pltpu.VMEM((eb, HIDW), jnp.float32),
        pltpu.VMEM((eb, ADW), jnp.float32),
        pltpu.VMEM((eb, ADW), jnp.float32),
        pltpu.VMEM((eb, ADW), jnp.float32),
        pltpu.VMEM((eb, ADW), jnp.float32),
        pltpu.VMEM((eb, ADW), jnp.float32),
        pltpu.VMEM((eb, ADW), jnp.float32),
        pltpu.VMEM((eb,), jnp.float32),
        pltpu.VMEM((eb,), jnp.float32),
        pltpu.VMEM((eb,), jnp.float32),
        pltpu.VMEM((ZROWS, HIDW), jnp.float32),
        pltpu.VMEM((HIDW,), jnp.float32),
        pltpu.SemaphoreType.DMA,
        pltpu.SemaphoreType.DMA,
        pltpu.SemaphoreType.DMA,
        pltpu.SemaphoreType.DMA,
        pltpu.SemaphoreType.DMA,
        pltpu.SemaphoreType.DMA,
        pltpu.SemaphoreType.DMA,
        pltpu.SemaphoreType.DMA,
        pltpu.SemaphoreType.DMA,
    ]


def _sc_layer1(tflat, astab, adtab, src, dst):
    mesh = plsc.VectorSubcoreMesh(core_axis_name="c", subcore_axis_name="s")

    @functools.partial(
        pl.kernel,
        out_type=[
            jax.ShapeDtypeStruct((NHEADS * NPAD, HIDW), jnp.float32),
            jax.ShapeDtypeStruct((NHEADS * NPAD,), jnp.float32),
        ],
        mesh=mesh,
        compiler_params=pltpu.CompilerParams(use_tc_tiling_on_sc=False,
                                             needs_layout_passes=False),
        scratch_types=_sc_scratch(EB1),
    )
    def body(t_hbm, as_hbm, ad_hbm, src_hbm, dst_hbm, acc_hbm, den_hbm,
             acc_sp, den_sp, s0, s1, s2, d0, d1, d2, x0, x1, x2, r0, r1, r2,
             a0, a1, a2, e0, e1, e2, w0, w1, w2, stage_v, dstage_v,
             si0, si1, si2, sg0, sg1, sg2, ss0, ss1, ss2):
        c = lax.axis_index("c")
        s = lax.axis_index("s")
        _zero_fill(stage_v, ZROWS)
        base_ctx = (t_hbm, as_hbm, ad_hbm, src_hbm, dst_hbm, acc_sp, den_sp,
                    (s0, s1, s2), (d0, d1, d2), (x0, x1, x2), (r0, r1, r2),
                    (a0, a1, a2), (e0, e1, e2), (w0, w1, w2),
                    (si0, si1, si2), (sg0, sg1, sg2), (ss0, ss1, ss2),
                    s * (EE // NTILE))
        for p in range(2):
            hd = 2 * c + p
            _acc_zero(acc_sp, den_sp, stage_v, s)
            plsc.subcore_barrier()
            _edge_pass(base_ctx, NB1, EB1, hd * NPAD, hd)
            plsc.subcore_barrier()
            _acc_readout(acc_sp, den_sp, stage_v, dstage_v, acc_hbm, den_hbm,
                         s, hd * NPAD)
            plsc.subcore_barrier()
            if p == 0:
                _zero_fill(stage_v, ZROWS)

    return body(tflat, astab, adtab, src, dst)


def _sc_layer2(tbl, astab, adtab, src, dst):
    mesh = plsc.VectorSubcoreMesh(core_axis_name="c", subcore_axis_name="s")

    @functools.partial(
        pl.kernel,
        out_type=[
            jax.ShapeDtypeStruct((NCORE * NPAD, HIDW), jnp.float32),
            jax.ShapeDtypeStruct((NCORE * NPAD,), jnp.float32),
        ],
        mesh=mesh,
        compiler_params=pltpu.CompilerParams(use_tc_tiling_on_sc=False,
                                             needs_layout_passes=False),
        scratch_types=_sc_scratch(EB2),
    )
    def body(t_hbm, as_hbm, ad_hbm, src_hbm, dst_hbm, acc_hbm, den_hbm,
             acc_sp, den_sp, s0, s1, s2, d0, d1, d2, x0, x1, x2, r0, r1, r2,
             a0, a1, a2, e0, e1, e2, w0, w1, w2, stage_v, dstage_v,
             si0, si1, si2, sg0, sg1, sg2, ss0, ss1, ss2):
        c = lax.axis_index("c")
        s = lax.axis_index("s")
        wid = c * NTILE + s
        _zero_fill(stage_v, ZROWS)
        _acc_zero(acc_sp, den_sp, stage_v, s)
        plsc.subcore_barrier()
        ctx = (t_hbm, as_hbm, ad_hbm, src_hbm, dst_hbm, acc_sp, den_sp,
               (s0, s1, s2), (d0, d1, d2), (x0, x1, x2), (r0, r1, r2),
               (a0, a1, a2), (e0, e1, e2), (w0, w1, w2),
               (si0, si1, si2), (sg0, sg1, sg2), (ss0, ss1, ss2),
               wid * (EE // (NTILE * NCORE)))
        _edge_pass(ctx, NB2, EB2, None, 0)
        plsc.subcore_barrier()
        _acc_readout(acc_sp, den_sp, stage_v, dstage_v, acc_hbm, den_hbm, s,
                     c * NPAD)

    return body(tbl, astab, adtab, src, dst)


def kernel(x, edge_index, W1, a1_src, a1_dst, b1, W2, a2_src, a2_dst, b2,
           Wout, bout):
    x_pad = jnp.pad(x, ((0, NPAD - NN), (0, 0)))
    src = jnp.pad(edge_index[0], (0, EIDX_PAD - EE))
    dst = jnp.pad(edge_index[1], (0, EIDX_PAD - EE))

    feats1, astab1, adtab1 = _l1_tables(x_pad, W1, a1_src, a1_dst)
    acc1, den1 = _sc_layer1(feats1.reshape(NHEADS * NPAD, HIDW), astab1,
                            adtab1, src, dst)
    acc1 = acc1.reshape(NHEADS, NPAD, HIDW)
    den1 = den1.reshape(NHEADS, NPAD, 1)

    feats2, astab2, adtab2 = _l2_tables(acc1, den1, b1.reshape(NHEADS, HIDW),
                                        W2, a2_src, a2_dst)
    acc2, den2 = _sc_layer2(feats2, astab2, adtab2, src, dst)
    acc2 = acc2.reshape(NCORE, NPAD, HIDW)
    den2 = den2.reshape(NCORE, NPAD, 1)

    y = _final(acc2, den2, b2.reshape(1, HIDW), Wout, bout.reshape(1, 1))
    return y[:NN]


# speedup vs baseline: 2.1667x; 1.0157x over previous
import functools

import jax
import jax.numpy as jnp
from jax import lax
from jax.experimental import pallas as pl
from jax.experimental.pallas import tpu as pltpu
from jax.experimental.pallas import tpu_sc as plsc

NN = 10000
EE = 160000
NPAD = 10240
DIN = 256
HIDW = 128
NHEADS = 4
ADW = 16
BLK = 256
NBLK = NPAD // BLK

NTILE = 16
NCORE = 2
ROWS_PER_TILE = NPAD // NTILE
EB1 = 80
NB1 = (EE // NTILE) // EB1
EB2 = 40
NB2 = (EE // (NTILE * NCORE)) // EB2
EIDX_PAD = EE + 4 * EB1
ZROWS = 40


def _l1_tables_body(x_ref, w1_ref, a1s_ref, a1d_ref, t_ref, as_ref, ad_ref):
    h = jnp.dot(x_ref[...], w1_ref[...], preferred_element_type=jnp.float32)
    as_ref[:, NHEADS:ADW] = jnp.zeros((BLK, ADW - NHEADS), jnp.float32)
    ad_ref[:, NHEADS:ADW] = jnp.zeros((BLK, ADW - NHEADS), jnp.float32)
    for hd in range(NHEADS):
        hh = h[:, hd * HIDW:(hd + 1) * HIDW]
        asv = jnp.sum(hh * a1s_ref[hd, :][None, :], axis=1, keepdims=True)
        adv = jnp.sum(hh * a1d_ref[hd, :][None, :], axis=1, keepdims=True)
        t_ref[hd] = hh
        as_ref[:, hd:hd + 1] = asv
        ad_ref[:, hd:hd + 1] = adv


def _l1_tables(x_pad, W1, a1_src, a1_dst):
    return pl.pallas_call(
        _l1_tables_body,
        grid=(NBLK,),
        in_specs=[
            pl.BlockSpec((BLK, DIN), lambda i: (i, 0)),
            pl.BlockSpec((DIN, NHEADS * HIDW), lambda i: (0, 0)),
            pl.BlockSpec((NHEADS, HIDW), lambda i: (0, 0)),
            pl.BlockSpec((NHEADS, HIDW), lambda i: (0, 0)),
        ],
        out_specs=[
            pl.BlockSpec((NHEADS, BLK, HIDW), lambda i: (0, i, 0)),
            pl.BlockSpec((BLK, ADW), lambda i: (i, 0)),
            pl.BlockSpec((BLK, ADW), lambda i: (i, 0)),
        ],
        out_shape=[
            jax.ShapeDtypeStruct((NHEADS, NPAD, HIDW), jnp.float32),
            jax.ShapeDtypeStruct((NPAD, ADW), jnp.float32),
            jax.ShapeDtypeStruct((NPAD, ADW), jnp.float32),
        ],
    )(x_pad, W1, a1_src, a1_dst)


def _l2_tables_body(acc_ref, den_ref, b1_ref, w2_ref, a2s_ref, a2d_ref,
                    t_ref, as_ref, ad_ref):
    hs = []
    for hd in range(NHEADS):
        v = acc_ref[hd] / (den_ref[hd] + 1e-16) + b1_ref[hd, :][None, :]
        hs.append(jnp.where(v > 0, v, jnp.exp(v) - 1.0))
    h1n = jnp.concatenate(hs, axis=1)
    h2 = jnp.dot(h1n, w2_ref[...], preferred_element_type=jnp.float32)
    asv = jnp.sum(h2 * a2s_ref[0, :][None, :], axis=1, keepdims=True)
    adv = jnp.sum(h2 * a2d_ref[0, :][None, :], axis=1, keepdims=True)
    t_ref[...] = h2
    as_ref[:, 0:1] = asv
    as_ref[:, 1:ADW] = jnp.zeros((BLK, ADW - 1), jnp.float32)
    ad_ref[:, 0:1] = adv
    ad_ref[:, 1:ADW] = jnp.zeros((BLK, ADW - 1), jnp.float32)


def _l2_tables(acc1, den1, b1r, W2, a2_src, a2_dst):
    return pl.pallas_call(
        _l2_tables_body,
        grid=(NBLK,),
        in_specs=[
            pl.BlockSpec((NHEADS, BLK, HIDW), lambda i: (0, i, 0)),
            pl.BlockSpec((NHEADS, BLK, 1), lambda i: (0, i, 0)),
            pl.BlockSpec((NHEADS, HIDW), lambda i: (0, 0)),
            pl.BlockSpec((NHEADS * HIDW, HIDW), lambda i: (0, 0)),
            pl.BlockSpec((1, HIDW), lambda i: (0, 0)),
            pl.BlockSpec((1, HIDW), lambda i: (0, 0)),
        ],
        out_specs=[
            pl.BlockSpec((BLK, HIDW), lambda i: (i, 0)),
            pl.BlockSpec((BLK, ADW), lambda i: (i, 0)),
            pl.BlockSpec((BLK, ADW), lambda i: (i, 0)),
        ],
        out_shape=[
            jax.ShapeDtypeStruct((NPAD, HIDW), jnp.float32),
            jax.ShapeDtypeStruct((NPAD, ADW), jnp.float32),
            jax.ShapeDtypeStruct((NPAD, ADW), jnp.float32),
        ],
    )(acc1, den1, b1r, W2, a2_src, a2_dst)


def _final_body(acc_ref, den_ref, b2_ref, wout_ref, bout_ref, o_ref):
    a = acc_ref[0] + acc_ref[1]
    d = den_ref[0] + den_ref[1]
    v = a / (d + 1e-16) + b2_ref[0, :][None, :]
    h = jnp.where(v > 0, v, jnp.exp(v) - 1.0)
    o_ref[...] = jnp.dot(h, wout_ref[...],
                         preferred_element_type=jnp.float32) + bout_ref[0, :][None, :]


def _final(acc2, den2, b2r, Wout, boutr):
    return pl.pallas_call(
        _final_body,
        grid=(NBLK,),
        in_specs=[
            pl.BlockSpec((NCORE, BLK, HIDW), lambda i: (0, i, 0)),
            pl.BlockSpec((NCORE, BLK, 1), lambda i: (0, i, 0)),
            pl.BlockSpec((1, HIDW), lambda i: (0, 0)),
            pl.BlockSpec((HIDW, 1), lambda i: (0, 0)),
            pl.BlockSpec((1, 1), lambda i: (0, 0)),
        ],
        out_specs=pl.BlockSpec((BLK, 1), lambda i: (i, 0)),
        out_shape=jax.ShapeDtypeStruct((NPAD, 1), jnp.float32),
    )(acc2, den2, b2r, Wout, boutr)


def _w_offsets(nb):
    offs = list(range(0, nb - 15, 16))
    if offs[-1] != nb - 16:
        offs.append(nb - 16)
    return offs


def _zero_fill(zero_v, nrows):
    zv = jnp.zeros((16,), jnp.float32)

    def zrow(j, _):
        for k in range(HIDW // 16):
            zero_v[j, pl.ds(k * 16, 16)] = zv
        return 0

    lax.fori_loop(0, nrows, zrow, 0)


def _scale_rows(rows, w_v, eb):
    def group(w16, r0, j0, nj):
        for j in range(j0, nj):
            wr = w16[j]
            for k in range(HIDW // 16):
                rows[r0 + j, pl.ds(k * 16, 16)] = (
                    rows[r0 + j, pl.ds(k * 16, 16)] * wr)

    def body(g, _):
        group(w_v[pl.ds(g * 16, 16)], g * 16, 0, 16)
        return 0

    lax.fori_loop(0, eb // 16, body, 0)
    rem = eb - (eb // 16) * 16
    if rem:
        lane0 = (eb // 16) * 16 - (eb - 16)
        w16 = w_v[pl.ds(eb - 16, 16)]
        r0 = (eb // 16) * 16
        for j in range(rem):
            wr = w16[lane0 + j]
            for k in range(HIDW // 16):
                rows[r0 + j, pl.ds(k * 16, 16)] = (
                    rows[r0 + j, pl.ds(k * 16, 16)] * wr)


def _issue_gathers(ctx, i, b, eb, row_off, adcol, wait_scatter):
    (t_hbm, as_hbm, ad_hbm, src_hbm, dst_hbm, acc_sp, den_sp, sbuf, dbuf,
     scat, rbuf, abuf, ebuf, wbuf, semi, semg, sems, tile_base) = ctx
    base = tile_base + i * eb
    pltpu.make_async_copy(src_hbm.at[pl.ds(base, eb)], sbuf[b], semi[b]).wait()
    pltpu.make_async_copy(dst_hbm.at[pl.ds(base, eb)], dbuf[b], semi[b]).wait()
    if wait_scatter:
        pltpu.make_async_copy(rbuf[b], acc_sp.at[scat[b]], sems[b]).wait()
        pltpu.make_async_copy(wbuf[b], den_sp.at[scat[b]], sems[b]).wait()
    pltpu.async_copy(as_hbm.at[sbuf[b]], abuf[b], semg[b])
    if row_off is not None:
        for off in range(0, eb, 16):
            sbuf[b][pl.ds(off, 16)] = sbuf[b][pl.ds(off, 16)] + row_off
    pltpu.async_copy(t_hbm.at[sbuf[b]], rbuf[b], semg[b])
    pltpu.async_copy(ad_hbm.at[dbuf[b]], ebuf[b], semg[b])


def _pipe_step(ctx, i, b, eb, row_off, adcol, first, second):
    (t_hbm, as_hbm, ad_hbm, src_hbm, dst_hbm, acc_sp, den_sp, sbuf, dbuf,
     scat, rbuf, abuf, ebuf, wbuf, semi, semg, sems, tile_base) = ctx
    b2 = (b + 2) % 3
    i16 = lax.iota(jnp.int32, 16)
    cas = i16 * 0 + adcol
    pltpu.make_async_copy(as_hbm.at[sbuf[b]], abuf[b], semg[b]).wait()
    pltpu.make_async_copy(t_hbm.at[sbuf[b]], rbuf[b], semg[b]).wait()
    pltpu.make_async_copy(ad_hbm.at[dbuf[b]], ebuf[b], semg[b]).wait()
    for off in _w_offsets(eb):
        scat[b][pl.ds(off, 16)] = dbuf[b][pl.ds(off, 16)]
    base3 = tile_base + (i + 3) * eb
    pltpu.async_copy(src_hbm.at[pl.ds(base3, eb)], sbuf[b], semi[b])
    pltpu.async_copy(dst_hbm.at[pl.ds(base3, eb)], dbuf[b], semi[b])
    for off in _w_offsets(eb):
        rr = i16 + off
        asg = plsc.load_gather(abuf[b], [rr, cas])
        adg = plsc.load_gather(ebuf[b], [rr, cas])
        e = asg + adg
        e = jnp.maximum(e, e * 0.2)
        wbuf[b][pl.ds(off, 16)] = jnp.exp(e)
    _scale_rows(rbuf[b], wbuf[b], eb)
    pltpu.async_copy(rbuf[b], acc_sp.at[scat[b]], sems[b], add=True)
    pltpu.async_copy(wbuf[b], den_sp.at[scat[b]], sems[b], add=True)
    _issue_gathers(ctx, i + 2, b2, eb, row_off, adcol, not first)


def _edge_pass(ctx, nb, eb, row_off, adcol):
    (t_hbm, as_hbm, ad_hbm, src_hbm, dst_hbm, acc_sp, den_sp, sbuf, dbuf,
     scat, rbuf, abuf, ebuf, wbuf, semi, semg, sems, tile_base) = ctx
    assert (nb - 2) % 3 == 0
    for j in range(3):
        base = tile_base + j * eb
        pltpu.async_copy(src_hbm.at[pl.ds(base, eb)], sbuf[j], semi[j])
        pltpu.async_copy(dst_hbm.at[pl.ds(base, eb)], dbuf[j], semi[j])
    _issue_gathers(ctx, 0, 0, eb, row_off, adcol, False)
    _issue_gathers(ctx, 1, 1, eb, row_off, adcol, False)
    _pipe_step(ctx, 0, 0, eb, row_off, adcol, True, False)
    _pipe_step(ctx, 1, 1, eb, row_off, adcol, False, True)

    def gbody(g, _):
        i = 3 * g + 2
        _pipe_step(ctx, i, 2, eb, row_off, adcol, False, False)
        _pipe_step(ctx, i + 1, 0, eb, row_off, adcol, False, False)
        _pipe_step(ctx, i + 2, 1, eb, row_off, adcol, False, False)
        return 0

    lax.fori_loop(0, (nb - 2) // 3, gbody, 0)
    for i in (nb, nb + 1):
        b = i % 3
        pltpu.make_async_copy(as_hbm.at[sbuf[b]], abuf[b], semg[b]).wait()
        pltpu.make_async_copy(t_hbm.at[sbuf[b]], rbuf[b], semg[b]).wait()
        pltpu.make_async_copy(ad_hbm.at[dbuf[b]], ebuf[b], semg[b]).wait()
    b = (nb - 1) % 3
    pltpu.make_async_copy(rbuf[b], acc_sp.at[scat[b]], sems[b]).wait()
    pltpu.make_async_copy(wbuf[b], den_sp.at[scat[b]], sems[b]).wait()
    i = nb + 2
    b = i % 3
    base = tile_base + i * eb
    pltpu.make_async_copy(src_hbm.at[pl.ds(base, eb)], sbuf[b], semi[b]).wait()
    pltpu.make_async_copy(dst_hbm.at[pl.ds(base, eb)], dbuf[b], semi[b]).wait()


def _acc_zero(acc_sp, den_sp, zero_v, s):
    def zacc(j, _):
        pltpu.sync_copy(zero_v,
                        acc_sp.at[pl.ds(s * ROWS_PER_TILE + j * ZROWS, ZROWS)])
        return 0
    lax.fori_loop(0, ROWS_PER_TILE // ZROWS, zacc, 0)

    def zden(j, _):
        pltpu.sync_copy(zero_v.at[0],
                        den_sp.at[pl.ds(s * ROWS_PER_TILE + j * HIDW, HIDW)])
        return 0
    lax.fori_loop(0, ROWS_PER_TILE // HIDW, zden, 0)


def _acc_readout(acc_sp, den_sp, stage_v, dstage_v, acc_hbm, den_hbm, s,
                 out_row_base):
    def rdout(j, _):
        rs = s * ROWS_PER_TILE + j * ZROWS
        pltpu.sync_copy(acc_sp.at[pl.ds(rs, ZROWS)], stage_v)
        pltpu.sync_copy(stage_v, acc_hbm.at[pl.ds(out_row_base + rs, ZROWS)])
        return 0
    lax.fori_loop(0, ROWS_PER_TILE // ZROWS, rdout, 0)

    def rden(j, _):
        rs = s * ROWS_PER_TILE + j * HIDW
        pltpu.sync_copy(den_sp.at[pl.ds(rs, HIDW)], dstage_v)
        pltpu.sync_copy(dstage_v, den_hbm.at[pl.ds(out_row_base + rs, HIDW)])
        return 0
    lax.fori_loop(0, ROWS_PER_TILE // HIDW, rden, 0)


def _sc_scratch(eb):
    return [
        pltpu.VMEM_SHARED((NPAD, HIDW), jnp.float32),
        pltpu.VMEM_SHARED((NPAD,), jnp.float32),
        pltpu.VMEM((eb,), jnp.int32),
        pltpu.VMEM((eb,), jnp.int32),
        pltpu.VMEM((eb,), jnp.int32),
        pltpu.VMEM((eb,), jnp.int32),
        pltpu.VMEM((eb,), jnp.int32),
        pltpu.VMEM((eb,), jnp.int32),
        pltpu.VMEM((eb,), jnp.int32),
        pltpu.VMEM((eb,), jnp.int32),
        pltpu.VMEM((eb,), jnp.int32),
        pltpu.VMEM((eb, HIDW), jnp.float32),
        pltpu.VMEM((eb, HIDW), jnp.float32),
        pltpu.VMEM((eb, HIDW), jnp.float32),
        pltpu.VMEM((eb, ADW), jnp.float32),
        pltpu.VMEM((eb, ADW), jnp.float32),
        pltpu.VMEM((eb, ADW), jnp.float32),
        pltpu.VMEM((eb, ADW), jnp.float32),
        pltpu.VMEM((eb, ADW), jnp.float32),
        pltpu.VMEM((eb, ADW), jnp.float32),
        pltpu.VMEM((eb,), jnp.float32),
        pltpu.VMEM((eb,), jnp.float32),
        pltpu.VMEM((eb,), jnp.float32),
        pltpu.VMEM((ZROWS, HIDW), jnp.float32),
        pltpu.VMEM((HIDW,), jnp.float32),
        pltpu.SemaphoreType.DMA,
        pltpu.SemaphoreType.DMA,
        pltpu.SemaphoreType.DMA,
        pltpu.SemaphoreType.DMA,
        pltpu.SemaphoreType.DMA,
        pltpu.SemaphoreType.DMA,
        pltpu.SemaphoreType.DMA,
        pltpu.SemaphoreType.DMA,
        pltpu.SemaphoreType.DMA,
    ]


def _sc_layer1(tflat, astab, adtab, src, dst):
    mesh = plsc.VectorSubcoreMesh(core_axis_name="c", subcore_axis_name="s")

    @functools.partial(
        pl.kernel,
        out_type=[
            jax.ShapeDtypeStruct((NHEADS * NPAD, HIDW), jnp.float32),
            jax.ShapeDtypeStruct((NHEADS * NPAD,), jnp.float32),
        ],
        mesh=mesh,
        compiler_params=pltpu.CompilerParams(use_tc_tiling_on_sc=False,
                                             needs_layout_passes=False),
        scratch_types=_sc_scratch(EB1),
    )
    def body(t_hbm, as_hbm, ad_hbm, src_hbm, dst_hbm, acc_hbm, den_hbm,
             acc_sp, den_sp, s0, s1, s2, d0, d1, d2, x0, x1, x2, r0, r1, r2,
             a0, a1, a2, e0, e1, e2, w0, w1, w2, stage_v, dstage_v,
             si0, si1, si2, sg0, sg1, sg2, ss0, ss1, ss2):
        c = lax.axis_index("c")
        s = lax.axis_index("s")
        _zero_fill(stage_v, ZROWS)
        base_ctx = (t_hbm, as_hbm, ad_hbm, src_hbm, dst_hbm, acc_sp, den_sp,
                    (s0, s1, s2), (d0, d1, d2), (x0, x1, x2), (r0, r1, r2),
                    (a0, a1, a2), (e0, e1, e2), (w0, w1, w2),
                    (si0, si1, si2), (sg0, sg1, sg2), (ss0, ss1, ss2),
                    s * (EE // NTILE))
        for p in range(2):
            hd = 2 * c + p
            _acc_zero(acc_sp, den_sp, stage_v, s)
            plsc.subcore_barrier()
            _edge_pass(base_ctx, NB1, EB1, hd * NPAD, hd)
            plsc.subcore_barrier()
            _acc_readout(acc_sp, den_sp, stage_v, dstage_v, acc_hbm, den_hbm,
                         s, hd * NPAD)
            plsc.subcore_barrier()
            if p == 0:
                _zero_fill(stage_v, ZROWS)

    return body(tflat, astab, adtab, src, dst)


def _sc_layer2(tbl, astab, adtab, src, dst):
    mesh = plsc.VectorSubcoreMesh(core_axis_name="c", subcore_axis_name="s")

    @functools.partial(
        pl.kernel,
        out_type=[
            jax.ShapeDtypeStruct((NCORE * NPAD, HIDW), jnp.float32),
            jax.ShapeDtypeStruct((NCORE * NPAD,), jnp.float32),
        ],
        mesh=mesh,
        compiler_params=pltpu.CompilerParams(use_tc_tiling_on_sc=False,
                                             needs_layout_passes=False),
        scratch_types=_sc_scratch(EB2),
    )
    def body(t_hbm, as_hbm, ad_hbm, src_hbm, dst_hbm, acc_hbm, den_hbm,
             acc_sp, den_sp, s0, s1, s2, d0, d1, d2, x0, x1, x2, r0, r1, r2,
             a0, a1, a2, e0, e1, e2, w0, w1, w2, stage_v, dstage_v,
             si0, si1, si2, sg0, sg1, sg2, ss0, ss1, ss2):
        c = lax.axis_index("c")
        s = lax.axis_index("s")
        wid = c * NTILE + s
        _zero_fill(stage_v, ZROWS)
        _acc_zero(acc_sp, den_sp, stage_v, s)
        plsc.subcore_barrier()
        ctx = (t_hbm, as_hbm, ad_hbm, src_hbm, dst_hbm, acc_sp, den_sp,
               (s0, s1, s2), (d0, d1, d2), (x0, x1, x2), (r0, r1, r2),
               (a0, a1, a2), (e0, e1, e2), (w0, w1, w2),
               (si0, si1, si2), (sg0, sg1, sg2), (ss0, ss1, ss2),
               wid * (EE // (NTILE * NCORE)))
        _edge_pass(ctx, NB2, EB2, None, 0)
        plsc.subcore_barrier()
        _acc_readout(acc_sp, den_sp, stage_v, dstage_v, acc_hbm, den_hbm, s,
                     c * NPAD)

    return body(tbl, astab, adtab, src, dst)


def kernel(x, edge_index, W1, a1_src, a1_dst, b1, W2, a2_src, a2_dst, b2,
           Wout, bout):
    x_pad = jnp.pad(x, ((0, NPAD - NN), (0, 0)))
    src = jnp.pad(edge_index[0], (0, EIDX_PAD - EE))
    dst = jnp.pad(edge_index[1], (0, EIDX_PAD - EE))

    feats1, astab1, adtab1 = _l1_tables(x_pad, W1, a1_src, a1_dst)
    acc1, den1 = _sc_layer1(feats1.reshape(NHEADS * NPAD, HIDW), astab1,
                            adtab1, src, dst)
    acc1 = acc1.reshape(NHEADS, NPAD, HIDW)
    den1 = den1.reshape(NHEADS, NPAD, 1)

    feats2, astab2, adtab2 = _l2_tables(acc1, den1, b1.reshape(NHEADS, HIDW),
                                        W2, a2_src, a2_dst)
    acc2, den2 = _sc_layer2(feats2, astab2, adtab2, src, dst)
    acc2 = acc2.reshape(NCORE, NPAD, HIDW)
    den2 = den2.reshape(NCORE, NPAD, 1)

    y = _final(acc2, den2, b2.reshape(1, HIDW), Wout, bout.reshape(1, 1))
    return y[:NN]
